# baseline (speedup 1.0000x reference)
import functools

import jax
import jax.numpy as jnp
from jax import lax
from jax.experimental import pallas as pl
from jax.experimental.pallas import tpu as pltpu
from jax.experimental.pallas import tpu_sc as plsc


def _zero_fill_kernel(out_ref):
    out_ref[...] = jnp.zeros_like(out_ref)


def _tc_fill(B, T, C, dtype):
    n_blocks = 4 if T % 4 == 0 else 1
    return pl.pallas_call(
        _zero_fill_kernel,
        grid=(n_blocks,),
        out_specs=pl.BlockSpec((B, T // n_blocks, C), lambda i: (0, i, 0)),
        out_shape=jax.ShapeDtypeStruct((B, T, C), dtype),
    )()


_ZWORDS = 8192


def _sc_fill(n_total):
    info = plsc.get_sparse_core_info()
    nc, ns = info.num_cores, info.num_subcores
    nw = nc * ns
    chunk = n_total // nw
    assert n_total % nw == 0 and chunk % _ZWORDS == 0
    n_dma = chunk // _ZWORDS
    mesh = plsc.VectorSubcoreMesh(core_axis_name="c", subcore_axis_name="s")

    @functools.partial(
        pl.kernel,
        mesh=mesh,
        out_type=jax.ShapeDtypeStruct((n_total,), jnp.float32),
        scratch_types=[
            pltpu.VMEM((_ZWORDS,), jnp.float32),
            pltpu.SemaphoreType.DMA,
        ],
    )
    def k(out_hbm, zbuf, sem):
        def init_body(i, carry):
            zbuf[pl.ds(i * 16, 16)] = jnp.zeros((16,), jnp.float32)
            return carry

        lax.fori_loop(0, _ZWORDS // 16, init_body, 0)
        wid = lax.axis_index("s") * nc + lax.axis_index("c")
        base = wid * chunk
        copies = [
            pltpu.async_copy(zbuf, out_hbm.at[pl.ds(base + j * _ZWORDS, _ZWORDS)], sem)
            for j in range(n_dma)
        ]
        for c in copies:
            c.wait()

    return k()


def kernel(x, params):
    B, T, C = x.shape
    return _sc_fill(B * T * C).reshape(B, T, C)


# speedup vs baseline: 1.0386x; 1.0386x over previous
import functools

import jax
import jax.numpy as jnp
from jax import lax
from jax.experimental import pallas as pl
from jax.experimental.pallas import tpu as pltpu
from jax.experimental.pallas import tpu_sc as plsc


def _zero_fill_kernel(out_ref):
    out_ref[...] = jnp.zeros_like(out_ref)


def _tc_fill(B, T, C, dtype):
    n_blocks = 4 if T % 4 == 0 else 1
    return pl.pallas_call(
        _zero_fill_kernel,
        grid=(n_blocks,),
        out_specs=pl.BlockSpec((B, T // n_blocks, C), lambda i: (0, i, 0)),
        out_shape=jax.ShapeDtypeStruct((B, T, C), dtype),
    )()


_ZWORDS = 2048


def _sc_fill(n_total):
    info = plsc.get_sparse_core_info()
    nc, ns = info.num_cores, info.num_subcores
    nw = nc * ns
    chunk = n_total // nw
    assert n_total % nw == 0 and chunk % _ZWORDS == 0
    n_dma = chunk // _ZWORDS
    mesh = plsc.VectorSubcoreMesh(core_axis_name="c", subcore_axis_name="s")

    @functools.partial(
        pl.kernel,
        mesh=mesh,
        out_type=jax.ShapeDtypeStruct((n_total,), jnp.float32),
        scratch_types=[
            pltpu.VMEM((_ZWORDS,), jnp.float32),
            pltpu.SemaphoreType.DMA,
        ],
    )
    def k(out_hbm, zbuf, sem):
        def init_body(i, carry):
            zbuf[pl.ds(i * 16, 16)] = jnp.zeros((16,), jnp.float32)
            return carry

        lax.fori_loop(0, _ZWORDS // 16, init_body, 0)
        wid = lax.axis_index("s") * nc + lax.axis_index("c")
        base = wid * chunk
        copies = [
            pltpu.async_copy(zbuf, out_hbm.at[pl.ds(base + j * _ZWORDS, _ZWORDS)], sem)
            for j in range(n_dma)
        ]
        for c in copies:
            c.wait()

    return k()


def kernel(x, params):
    B, T, C = x.shape
    return _sc_fill(B * T * C).reshape(B, T, C)


# speedup vs baseline: 8.4001x; 8.0875x over previous
import functools

import jax
import jax.numpy as jnp
from jax import lax
from jax.experimental import pallas as pl
from jax.experimental.pallas import tpu as pltpu
from jax.experimental.pallas import tpu_sc as plsc


def _zero_fill_kernel(out_ref):
    out_ref[...] = jnp.zeros_like(out_ref)


def _tc_fill(B, T, C, dtype):
    n_blocks = 4 if T % 4 == 0 else 1
    return pl.pallas_call(
        _zero_fill_kernel,
        grid=(n_blocks,),
        out_specs=pl.BlockSpec((B, T // n_blocks, C), lambda i: (0, i, 0)),
        out_shape=jax.ShapeDtypeStruct((B, T, C), dtype),
    )()


_ZWORDS = 2048


def _sc_fill(n_total):
    info = plsc.get_sparse_core_info()
    nc, ns = info.num_cores, info.num_subcores
    nw = nc * ns
    chunk = n_total // nw
    assert n_total % nw == 0 and chunk % _ZWORDS == 0
    n_dma = chunk // _ZWORDS
    mesh = plsc.VectorSubcoreMesh(core_axis_name="c", subcore_axis_name="s")

    @functools.partial(
        pl.kernel,
        mesh=mesh,
        out_type=jax.ShapeDtypeStruct((n_total,), jnp.float32),
        scratch_types=[
            pltpu.VMEM((_ZWORDS,), jnp.float32),
            pltpu.SemaphoreType.DMA,
        ],
    )
    def k(out_hbm, zbuf, sem):
        def init_body(i, carry):
            zbuf[pl.ds(i * 16, 16)] = jnp.zeros((16,), jnp.float32)
            return carry

        lax.fori_loop(0, _ZWORDS // 16, init_body, 0)
        wid = lax.axis_index("s") * nc + lax.axis_index("c")
        base = wid * chunk
        copies = [
            pltpu.async_copy(zbuf, out_hbm.at[pl.ds(base + j * _ZWORDS, _ZWORDS)], sem)
            for j in range(n_dma)
        ]
        for c in copies:
            c.wait()

    return k()


def _tc_fill_manual(B, T, C, dtype, rows=512):
    n_rows = B * T
    n_dma = n_rows // rows
    assert n_rows % rows == 0

    def body(out_ref, zbuf, sem):
        zbuf[...] = jnp.zeros_like(zbuf)
        copies = [
            pltpu.make_async_copy(zbuf, out_ref.at[pl.ds(i * rows, rows), :], sem)
            for i in range(n_dma)
        ]
        for c in copies:
            c.start()
        for c in copies:
            c.wait()

    out_flat = pl.pallas_call(
        body,
        out_specs=pl.BlockSpec(memory_space=pl.ANY),
        out_shape=jax.ShapeDtypeStruct((n_rows, C), dtype),
        scratch_shapes=[pltpu.VMEM((rows, C), dtype), pltpu.SemaphoreType.DMA],
    )()
    return out_flat.reshape(B, T, C)


def kernel(x, params):
    B, T, C = x.shape
    return _tc_fill_manual(B, T, C, x.dtype)


# speedup vs baseline: 8.8604x; 1.0548x over previous
import functools

import jax
import jax.numpy as jnp
from jax import lax
from jax.experimental import pallas as pl
from jax.experimental.pallas import tpu as pltpu
from jax.experimental.pallas import tpu_sc as plsc


def _zero_fill_kernel(out_ref):
    out_ref[...] = jnp.zeros_like(out_ref)


def _tc_fill(B, T, C, dtype):
    n_blocks = 4 if T % 4 == 0 else 1
    return pl.pallas_call(
        _zero_fill_kernel,
        grid=(n_blocks,),
        out_specs=pl.BlockSpec((B, T // n_blocks, C), lambda i: (0, i, 0)),
        out_shape=jax.ShapeDtypeStruct((B, T, C), dtype),
    )()


_ZWORDS = 2048


def _sc_fill(n_total):
    info = plsc.get_sparse_core_info()
    nc, ns = info.num_cores, info.num_subcores
    nw = nc * ns
    chunk = n_total // nw
    assert n_total % nw == 0 and chunk % _ZWORDS == 0
    n_dma = chunk // _ZWORDS
    mesh = plsc.VectorSubcoreMesh(core_axis_name="c", subcore_axis_name="s")

    @functools.partial(
        pl.kernel,
        mesh=mesh,
        out_type=jax.ShapeDtypeStruct((n_total,), jnp.float32),
        scratch_types=[
            pltpu.VMEM((_ZWORDS,), jnp.float32),
            pltpu.SemaphoreType.DMA,
        ],
    )
    def k(out_hbm, zbuf, sem):
        def init_body(i, carry):
            zbuf[pl.ds(i * 16, 16)] = jnp.zeros((16,), jnp.float32)
            return carry

        lax.fori_loop(0, _ZWORDS // 16, init_body, 0)
        wid = lax.axis_index("s") * nc + lax.axis_index("c")
        base = wid * chunk
        copies = [
            pltpu.async_copy(zbuf, out_hbm.at[pl.ds(base + j * _ZWORDS, _ZWORDS)], sem)
            for j in range(n_dma)
        ]
        for c in copies:
            c.wait()

    return k()


def _tc_fill_manual(B, T, C, dtype, rows=512):
    n_rows = B * T
    n_dma = n_rows // rows
    assert n_rows % rows == 0

    def body(out_ref, zbuf, sem):
        zbuf[...] = jnp.zeros_like(zbuf)
        copies = [
            pltpu.make_async_copy(zbuf, out_ref.at[pl.ds(i * rows, rows), :], sem)
            for i in range(n_dma)
        ]
        for c in copies:
            c.start()
        for c in copies:
            c.wait()

    out_flat = pl.pallas_call(
        body,
        out_specs=pl.BlockSpec(memory_space=pl.ANY),
        out_shape=jax.ShapeDtypeStruct((n_rows, C), dtype),
        scratch_shapes=[pltpu.VMEM((rows, C), dtype), pltpu.SemaphoreType.DMA],
    )()
    return out_flat.reshape(B, T, C)


_TC_TILE = 131072
_SC_TILE = 2048


def _hybrid_fill(n_total):
    info = plsc.get_sparse_core_info()
    nc, ns = info.num_cores, info.num_subcores
    nw = nc * ns
    sc_chunk = 6 * _SC_TILE
    sc_words = nw * sc_chunk
    tc_words = n_total - sc_words
    assert tc_words % _TC_TILE == 0 and tc_words % 8 == 0
    n_tc_dma = tc_words // _TC_TILE
    n_sc_dma = sc_chunk // _SC_TILE

    tc_mesh = pltpu.create_tensorcore_mesh("tc")
    sc_mesh = plsc.VectorSubcoreMesh(core_axis_name="c", subcore_axis_name="s")

    def tc_body(out_hbm, tc_zbuf, tc_sem, sc_zbuf, sc_sem):
        del sc_zbuf, sc_sem
        tc_zbuf[...] = jnp.zeros_like(tc_zbuf)
        copies = [
            pltpu.make_async_copy(
                tc_zbuf, out_hbm.at[pl.ds(i * _TC_TILE, _TC_TILE)], tc_sem)
            for i in range(n_tc_dma)
        ]
        for c in copies:
            c.start()
        for c in copies:
            c.wait()

    def sc_body(out_hbm, tc_zbuf, tc_sem, sc_zbuf, sc_sem):
        del tc_zbuf, tc_sem

        def init_body(i, carry):
            sc_zbuf[pl.ds(i * 16, 16)] = jnp.zeros((16,), jnp.float32)
            return carry

        lax.fori_loop(0, _SC_TILE // 16, init_body, 0)
        wid = lax.axis_index("s") * nc + lax.axis_index("c")
        base = tc_words + wid * sc_chunk
        copies = [
            pltpu.async_copy(
                sc_zbuf, out_hbm.at[pl.ds(base + j * _SC_TILE, _SC_TILE)], sc_sem)
            for j in range(n_sc_dma)
        ]
        for c in copies:
            c.wait()

    run = pl.kernel(
        [sc_body, tc_body],
        mesh=[sc_mesh, tc_mesh],
        out_type=pltpu.HBM((n_total,), jnp.float32),
        scratch_types=[
            (pltpu.MemorySpace.VMEM @ tc_mesh)((_TC_TILE,), jnp.float32),
            pltpu.SemaphoreType.DMA @ tc_mesh,
            (pltpu.MemorySpace.VMEM @ sc_mesh)((_SC_TILE,), jnp.float32),
            pltpu.SemaphoreType.DMA @ sc_mesh,
        ],
    )
    return run()


def kernel(x, params):
    B, T, C = x.shape
    return _tc_fill_manual(B, T, C, x.dtype, rows=128)


# speedup vs baseline: 8.8776x; 1.0019x over previous
import functools

import jax
import jax.numpy as jnp
from jax import lax
from jax.experimental import pallas as pl
from jax.experimental.pallas import tpu as pltpu
from jax.experimental.pallas import tpu_sc as plsc


def _zero_fill_kernel(out_ref):
    out_ref[...] = jnp.zeros_like(out_ref)


def _tc_fill(B, T, C, dtype):
    n_blocks = 4 if T % 4 == 0 else 1
    return pl.pallas_call(
        _zero_fill_kernel,
        grid=(n_blocks,),
        out_specs=pl.BlockSpec((B, T // n_blocks, C), lambda i: (0, i, 0)),
        out_shape=jax.ShapeDtypeStruct((B, T, C), dtype),
    )()


_ZWORDS = 2048


def _sc_fill(n_total):
    info = plsc.get_sparse_core_info()
    nc, ns = info.num_cores, info.num_subcores
    nw = nc * ns
    chunk = n_total // nw
    assert n_total % nw == 0 and chunk % _ZWORDS == 0
    n_dma = chunk // _ZWORDS
    mesh = plsc.VectorSubcoreMesh(core_axis_name="c", subcore_axis_name="s")

    @functools.partial(
        pl.kernel,
        mesh=mesh,
        out_type=jax.ShapeDtypeStruct((n_total,), jnp.float32),
        scratch_types=[
            pltpu.VMEM((_ZWORDS,), jnp.float32),
            pltpu.SemaphoreType.DMA,
        ],
    )
    def k(out_hbm, zbuf, sem):
        def init_body(i, carry):
            zbuf[pl.ds(i * 16, 16)] = jnp.zeros((16,), jnp.float32)
            return carry

        lax.fori_loop(0, _ZWORDS // 16, init_body, 0)
        wid = lax.axis_index("s") * nc + lax.axis_index("c")
        base = wid * chunk
        copies = [
            pltpu.async_copy(zbuf, out_hbm.at[pl.ds(base + j * _ZWORDS, _ZWORDS)], sem)
            for j in range(n_dma)
        ]
        for c in copies:
            c.wait()

    return k()


def _tc_fill_manual(B, T, C, dtype, rows=512):
    n_rows = B * T
    n_dma = n_rows // rows
    assert n_rows % rows == 0

    def body(out_ref, zbuf, sem):
        zbuf[...] = jnp.zeros_like(zbuf)
        copies = [
            pltpu.make_async_copy(zbuf, out_ref.at[pl.ds(i * rows, rows), :], sem)
            for i in range(n_dma)
        ]
        for c in copies:
            c.start()
        for c in copies:
            c.wait()

    out_flat = pl.pallas_call(
        body,
        out_specs=pl.BlockSpec(memory_space=pl.ANY),
        out_shape=jax.ShapeDtypeStruct((n_rows, C), dtype),
        scratch_shapes=[pltpu.VMEM((rows, C), dtype), pltpu.SemaphoreType.DMA],
    )()
    return out_flat.reshape(B, T, C)


_TC_TILE = 131072
_SC_TILE = 2048


def _hybrid_fill(n_total):
    info = plsc.get_sparse_core_info()
    nc, ns = info.num_cores, info.num_subcores
    nw = nc * ns
    sc_chunk = 6 * _SC_TILE
    sc_words = nw * sc_chunk
    tc_words = n_total - sc_words
    assert tc_words % _TC_TILE == 0 and tc_words % 8 == 0
    n_tc_dma = tc_words // _TC_TILE
    n_sc_dma = sc_chunk // _SC_TILE

    tc_mesh = pltpu.create_tensorcore_mesh("tc")
    sc_mesh = plsc.VectorSubcoreMesh(core_axis_name="c", subcore_axis_name="s")

    def tc_body(out_hbm, tc_zbuf, tc_sem, sc_zbuf, sc_sem):
        del sc_zbuf, sc_sem
        tc_zbuf[...] = jnp.zeros_like(tc_zbuf)
        copies = [
            pltpu.make_async_copy(
                tc_zbuf, out_hbm.at[pl.ds(i * _TC_TILE, _TC_TILE)], tc_sem)
            for i in range(n_tc_dma)
        ]
        for c in copies:
            c.start()
        for c in copies:
            c.wait()

    def sc_body(out_hbm, tc_zbuf, tc_sem, sc_zbuf, sc_sem):
        del tc_zbuf, tc_sem

        def init_body(i, carry):
            sc_zbuf[pl.ds(i * 16, 16)] = jnp.zeros((16,), jnp.float32)
            return carry

        lax.fori_loop(0, _SC_TILE // 16, init_body, 0)
        wid = lax.axis_index("s") * nc + lax.axis_index("c")
        base = tc_words + wid * sc_chunk
        copies = [
            pltpu.async_copy(
                sc_zbuf, out_hbm.at[pl.ds(base + j * _SC_TILE, _SC_TILE)], sc_sem)
            for j in range(n_sc_dma)
        ]
        for c in copies:
            c.wait()

    run = pl.kernel(
        [sc_body, tc_body],
        mesh=[sc_mesh, tc_mesh],
        out_type=pltpu.HBM((n_total,), jnp.float32),
        scratch_types=[
            (pltpu.MemorySpace.VMEM @ tc_mesh)((_TC_TILE,), jnp.float32),
            pltpu.SemaphoreType.DMA @ tc_mesh,
            (pltpu.MemorySpace.VMEM @ sc_mesh)((_SC_TILE,), jnp.float32),
            pltpu.SemaphoreType.DMA @ sc_mesh,
        ],
    )
    return run()


def kernel(x, params):
    B, T, C = x.shape
    return _tc_fill_manual(B, T, C, x.dtype, rows=64)


# speedup vs baseline: 8.9175x; 1.0045x over previous
import jax
import jax.numpy as jnp
from jax.experimental import pallas as pl
from jax.experimental.pallas import tpu as pltpu


def _make_fill_body(n_dma, rows):
    def body(out_ref, zbuf, sem):
        zbuf[...] = jnp.zeros_like(zbuf)
        copies = [
            pltpu.make_async_copy(zbuf, out_ref.at[pl.ds(i * rows, rows), :], sem)
            for i in range(n_dma)
        ]
        for c in copies:
            c.start()
        for c in copies:
            c.wait()

    return body


def kernel(x, params):
    B, T, C = x.shape
    n_rows = B * T
    rows = 64
    while n_rows % rows:
        rows //= 2
    out_flat = pl.pallas_call(
        _make_fill_body(n_rows // rows, rows),
        out_specs=pl.BlockSpec(memory_space=pl.ANY),
        out_shape=jax.ShapeDtypeStruct((n_rows, C), x.dtype),
        scratch_shapes=[pltpu.VMEM((rows, C), x.dtype), pltpu.SemaphoreType.DMA],
    )()
    return out_flat.reshape(B, T, C)
